# baseline (speedup 1.0000x reference)
import jax
import jax.numpy as jnp
import numpy as np
from jax.experimental import pallas as pl
from jax.experimental.pallas import tpu as pltpu

D = 768
H = 12
HD = D // H
NPAIR = H // 2
QBLK = 256


def _dot(a, b, dims):
    return jax.lax.dot_general(a, b, (dims, ((), ())),
                               preferred_element_type=jnp.float32)


def _causal_head_attn(q, k, v):
    n = q.shape[0]
    nblk = n // QBLK
    row = jax.lax.broadcasted_iota(jnp.int32, (QBLK, QBLK), 0)
    col = jax.lax.broadcasted_iota(jnp.int32, (QBLK, QBLK), 1)
    diag_mask = col <= row

    blocks = []
    for i in range(nblk):
        qi = q[i * QBLK:(i + 1) * QBLK, :]
        kd = k[i * QBLK:(i + 1) * QBLK, :]
        vd = v[i * QBLK:(i + 1) * QBLK, :]
        s_diag = _dot(qi, kd, ((1,), (1,)))
        s_diag = jnp.where(diag_mask, s_diag, -jnp.inf)
        if i == 0:
            m = jnp.max(s_diag, axis=1, keepdims=True)
            p_diag = jnp.exp(s_diag - m).astype(jnp.bfloat16)
            l = jnp.sum(p_diag.astype(jnp.float32), axis=1, keepdims=True)
            acc = _dot(p_diag, vd, ((1,), (0,)))
        else:
            kp = k[:i * QBLK, :]
            vp = v[:i * QBLK, :]
            s_prev = _dot(qi, kp, ((1,), (1,)))
            m = jnp.maximum(jnp.max(s_prev, axis=1, keepdims=True),
                            jnp.max(s_diag, axis=1, keepdims=True))
            p_prev = jnp.exp(s_prev - m).astype(jnp.bfloat16)
            p_diag = jnp.exp(s_diag - m).astype(jnp.bfloat16)
            l = (jnp.sum(p_prev.astype(jnp.float32), axis=1, keepdims=True) +
                 jnp.sum(p_diag.astype(jnp.float32), axis=1, keepdims=True))
            acc = (_dot(p_prev, vp, ((1,), (0,))) +
                   _dot(p_diag, vd, ((1,), (0,))))
        blocks.append(acc / l)
    return jnp.concatenate(blocks, axis=0)


def _mha_gate_kernel(x_ref, wall_ref, wot_ref, wg_ref, bknn_ref, bg_ref,
                     out_ref, qkv_ref):
    t = pl.program_id(0)

    @pl.when(t == 0)
    def _():
        qkv = _dot(x_ref[...], wall_ref[...], ((1,), (1,)))
        scale = jnp.concatenate(
            [jnp.full((1, D), 1.0 / np.sqrt(HD), jnp.float32),
             jnp.ones((1, 2 * D), jnp.float32)], axis=1)
        qkv_ref[...] = (qkv * scale).astype(jnp.bfloat16)

    @pl.when(t > 0)
    def _():
        pair = t - 1
        base = pair * (2 * HD)
        q2 = qkv_ref[:, pl.ds(base, 2 * HD)]
        k2 = qkv_ref[:, pl.ds(D + base, 2 * HD)]
        v2 = qkv_ref[:, pl.ds(2 * D + base, 2 * HD)]

        a0 = _causal_head_attn(q2[:, :HD], k2[:, :HD], v2[:, :HD])
        a1 = _causal_head_attn(q2[:, HD:], k2[:, HD:], v2[:, HD:])
        attn_pair = jnp.concatenate([a0, a1], axis=1).astype(jnp.bfloat16)
        wot_pair = wot_ref[pl.ds(base, 2 * HD), :]
        o = _dot(attn_pair, wot_pair, ((1,), (0,)))

        @pl.when(t == 1)
        def _():
            out_ref[...] = o

        @pl.when(t > 1)
        def _():
            out_ref[...] += o

        @pl.when(t == NPAIR)
        def _():
            a = out_ref[...]
            wg = wg_ref[...]
            wg1 = wg[:, :D]
            wg2 = wg[:, D:]
            bknn = bknn_ref[...]
            c = jnp.sum(bknn * wg2) + bg_ref[0, 0]
            g_logit = jnp.sum(a * wg1, axis=1, keepdims=True) + c
            g = jax.nn.sigmoid(g_logit)
            out_ref[...] = g * a + (1.0 - g) * bknn


def kernel(x, Wq, Wk, Wv, Wo, Wknn_key, Wknn_proj, b_knn_proj, Wgate, b_gate):
    b, s, d = x.shape
    x2 = x.reshape(s, d).astype(jnp.bfloat16)
    w_all = jnp.concatenate([Wq, Wk, Wv], axis=0).astype(jnp.bfloat16)
    wot = Wo.T.astype(jnp.bfloat16)
    bknn = b_knn_proj.reshape(1, d)
    bg = b_gate.reshape(1, 1)

    out = pl.pallas_call(
        _mha_gate_kernel,
        grid=(1 + NPAIR,),
        in_specs=[
            pl.BlockSpec((s, d), lambda t: (0, 0)),
            pl.BlockSpec((3 * d, d), lambda t: (0, 0)),
            pl.BlockSpec((d, d), lambda t: (0, 0)),
            pl.BlockSpec((1, 2 * d), lambda t: (0, 0)),
            pl.BlockSpec((1, d), lambda t: (0, 0)),
            pl.BlockSpec((1, 1), lambda t: (0, 0)),
        ],
        out_specs=pl.BlockSpec((s, d), lambda t: (0, 0)),
        out_shape=jax.ShapeDtypeStruct((s, d), jnp.float32),
        scratch_shapes=[pltpu.VMEM((s, 3 * d), jnp.bfloat16)],
        compiler_params=pltpu.CompilerParams(
            dimension_semantics=("arbitrary",),
        ),
    )(x2, w_all, wot, Wgate, bknn, bg)
    return out.reshape(b, s, d)


# speedup vs baseline: 1.1589x; 1.1589x over previous
import jax
import jax.numpy as jnp
import numpy as np
from jax.experimental import pallas as pl
from jax.experimental.pallas import tpu as pltpu

D = 768
H = 12
HD = D // H
NPAIR = H // 2
QBLK = 256


def _dot(a, b, dims):
    return jax.lax.dot_general(a, b, (dims, ((), ())),
                               preferred_element_type=jnp.float32)


def _causal_head_attn(q, k, v):
    n = q.shape[0]
    nblk = n // QBLK
    row = jax.lax.broadcasted_iota(jnp.int32, (QBLK, QBLK), 0)
    col = jax.lax.broadcasted_iota(jnp.int32, (QBLK, QBLK), 1)
    diag_mask = col <= row

    blocks = []
    for i in range(nblk):
        qi = q[i * QBLK:(i + 1) * QBLK, :]
        kd = k[i * QBLK:(i + 1) * QBLK, :]
        vd = v[i * QBLK:(i + 1) * QBLK, :]
        s_diag = _dot(qi, kd, ((1,), (1,)))
        s_diag = jnp.where(diag_mask, s_diag, -jnp.inf)
        if i == 0:
            m = jnp.max(s_diag, axis=1, keepdims=True)
            p_diag = jnp.exp(s_diag - m)
            l = jnp.sum(p_diag, axis=1, keepdims=True)
            acc = _dot(p_diag, vd, ((1,), (0,)))
        else:
            kp = k[:i * QBLK, :]
            vp = v[:i * QBLK, :]
            s_prev = _dot(qi, kp, ((1,), (1,)))
            m = jnp.maximum(jnp.max(s_prev, axis=1, keepdims=True),
                            jnp.max(s_diag, axis=1, keepdims=True))
            p_prev = jnp.exp(s_prev - m)
            p_diag = jnp.exp(s_diag - m)
            l = (jnp.sum(p_prev, axis=1, keepdims=True) +
                 jnp.sum(p_diag, axis=1, keepdims=True))
            acc = (_dot(p_prev, vp, ((1,), (0,))) +
                   _dot(p_diag, vd, ((1,), (0,))))
        blocks.append(acc / l)
    return jnp.concatenate(blocks, axis=0)


def _mha_gate_kernel(x_ref, w_ref, wot_ref, wg_ref, bknn_ref, bg_ref,
                     out_ref):
    t = pl.program_id(0)

    qkv = _dot(x_ref[...], w_ref[...], ((1,), (1,)))
    q2 = qkv[:, :2 * HD] * (1.0 / np.sqrt(HD))
    k2 = qkv[:, 2 * HD:4 * HD]
    v2 = qkv[:, 4 * HD:]

    a0 = _causal_head_attn(q2[:, :HD], k2[:, :HD], v2[:, :HD])
    a1 = _causal_head_attn(q2[:, HD:], k2[:, HD:], v2[:, HD:])
    attn_pair = jnp.concatenate([a0, a1], axis=1)
    o = _dot(attn_pair, wot_ref[...], ((1,), (0,)))

    @pl.when(t == 0)
    def _():
        out_ref[...] = o

    @pl.when(t > 0)
    def _():
        out_ref[...] += o

    @pl.when(t == NPAIR - 1)
    def _():
        a = out_ref[...]
        wg = wg_ref[...]
        wg1 = wg[:, :D]
        wg2 = wg[:, D:]
        bknn = bknn_ref[...]
        c = jnp.sum(bknn * wg2) + bg_ref[0, 0]
        g_logit = jnp.sum(a * wg1, axis=1, keepdims=True) + c
        g = jax.nn.sigmoid(g_logit)
        out_ref[...] = g * a + (1.0 - g) * bknn


def kernel(x, Wq, Wk, Wv, Wo, Wknn_key, Wknn_proj, b_knn_proj, Wgate, b_gate):
    b, s, d = x.shape
    x2 = x.reshape(s, d)
    w3 = jnp.stack([Wq, Wk, Wv], axis=0)
    w3 = w3.reshape(3, NPAIR, 2 * HD, d)
    w_pairs = w3.transpose(1, 0, 2, 3).reshape(NPAIR * 6 * HD, d)
    wot = Wo.T
    bknn = b_knn_proj.reshape(1, d)
    bg = b_gate.reshape(1, 1)

    out = pl.pallas_call(
        _mha_gate_kernel,
        grid=(NPAIR,),
        in_specs=[
            pl.BlockSpec((s, d), lambda t: (0, 0)),
            pl.BlockSpec((6 * HD, d), lambda t: (t, 0)),
            pl.BlockSpec((2 * HD, d), lambda t: (t, 0)),
            pl.BlockSpec((1, 2 * d), lambda t: (0, 0)),
            pl.BlockSpec((1, d), lambda t: (0, 0)),
            pl.BlockSpec((1, 1), lambda t: (0, 0)),
        ],
        out_specs=pl.BlockSpec((s, d), lambda t: (0, 0)),
        out_shape=jax.ShapeDtypeStruct((s, d), jnp.float32),
        compiler_params=pltpu.CompilerParams(
            dimension_semantics=("arbitrary",),
        ),
    )(x2, w_pairs, wot, Wgate, bknn, bg)
    return out.reshape(b, s, d)


# speedup vs baseline: 1.2080x; 1.0424x over previous
import jax
import jax.numpy as jnp
import numpy as np
from jax.experimental import pallas as pl
from jax.experimental.pallas import tpu as pltpu

D = 768
H = 12
HD = D // H
GH = 4
NSTEP = H // GH
QBLK = 256


def _dot(a, b, dims):
    return jax.lax.dot_general(a, b, (dims, ((), ())),
                               preferred_element_type=jnp.float32)


def _causal_head_attn(q, k, v):
    n = q.shape[0]
    nblk = n // QBLK
    row = jax.lax.broadcasted_iota(jnp.int32, (QBLK, QBLK), 0)
    col = jax.lax.broadcasted_iota(jnp.int32, (QBLK, QBLK), 1)
    diag_mask = col <= row

    blocks = []
    for i in range(nblk):
        qi = q[i * QBLK:(i + 1) * QBLK, :]
        kd = k[i * QBLK:(i + 1) * QBLK, :]
        vd = v[i * QBLK:(i + 1) * QBLK, :]
        s_diag = _dot(qi, kd, ((1,), (1,)))
        s_diag = jnp.where(diag_mask, s_diag, -jnp.inf)
        if i == 0:
            m = jnp.max(s_diag, axis=1, keepdims=True)
            p_diag = jnp.exp(s_diag - m)
            l = jnp.sum(p_diag, axis=1, keepdims=True)
            acc = _dot(p_diag, vd, ((1,), (0,)))
        else:
            kp = k[:i * QBLK, :]
            vp = v[:i * QBLK, :]
            s_prev = _dot(qi, kp, ((1,), (1,)))
            m = jnp.maximum(jnp.max(s_prev, axis=1, keepdims=True),
                            jnp.max(s_diag, axis=1, keepdims=True))
            p_prev = jnp.exp(s_prev - m)
            p_diag = jnp.exp(s_diag - m)
            l = (jnp.sum(p_prev, axis=1, keepdims=True) +
                 jnp.sum(p_diag, axis=1, keepdims=True))
            acc = (_dot(p_prev, vp, ((1,), (0,))) +
                   _dot(p_diag, vd, ((1,), (0,))))
        blocks.append(acc / l)
    return jnp.concatenate(blocks, axis=0)


def _mha_gate_kernel(x_ref, w_ref, wot_ref, wg_ref, bknn_ref, bg_ref,
                     out_ref):
    t = pl.program_id(0)
    gw = GH * HD

    qkv = _dot(x_ref[...], w_ref[...], ((1,), (1,)))
    scale = 1.0 / np.sqrt(HD)

    heads = []
    for g in range(GH):
        q = qkv[:, g * HD:(g + 1) * HD] * scale
        k = qkv[:, gw + g * HD:gw + (g + 1) * HD]
        v = qkv[:, 2 * gw + g * HD:2 * gw + (g + 1) * HD]
        heads.append(_causal_head_attn(q, k, v))
    attn_grp = jnp.concatenate(heads, axis=1)
    o = _dot(attn_grp, wot_ref[...], ((1,), (0,)))

    @pl.when(t == 0)
    def _():
        out_ref[...] = o

    @pl.when(t > 0)
    def _():
        out_ref[...] += o

    @pl.when(t == NSTEP - 1)
    def _():
        a = out_ref[...]
        wg = wg_ref[...]
        wg1 = wg[:, :D]
        wg2 = wg[:, D:]
        bknn = bknn_ref[...]
        c = jnp.sum(bknn * wg2) + bg_ref[0, 0]
        g_logit = jnp.sum(a * wg1, axis=1, keepdims=True) + c
        g = jax.nn.sigmoid(g_logit)
        out_ref[...] = g * a + (1.0 - g) * bknn


def kernel(x, Wq, Wk, Wv, Wo, Wknn_key, Wknn_proj, b_knn_proj, Wgate, b_gate):
    b, s, d = x.shape
    x2 = x.reshape(s, d)
    w3 = jnp.stack([Wq, Wk, Wv], axis=0)
    w3 = w3.reshape(3, NSTEP, GH * HD, d)
    w_groups = w3.transpose(1, 0, 2, 3).reshape(NSTEP * 3 * GH * HD, d)
    wot = Wo.T
    bknn = b_knn_proj.reshape(1, d)
    bg = b_gate.reshape(1, 1)

    out = pl.pallas_call(
        _mha_gate_kernel,
        grid=(NSTEP,),
        in_specs=[
            pl.BlockSpec((s, d), lambda t: (0, 0)),
            pl.BlockSpec((3 * GH * HD, d), lambda t: (t, 0)),
            pl.BlockSpec((GH * HD, d), lambda t: (t, 0)),
            pl.BlockSpec((1, 2 * d), lambda t: (0, 0)),
            pl.BlockSpec((1, d), lambda t: (0, 0)),
            pl.BlockSpec((1, 1), lambda t: (0, 0)),
        ],
        out_specs=pl.BlockSpec((s, d), lambda t: (0, 0)),
        out_shape=jax.ShapeDtypeStruct((s, d), jnp.float32),
        compiler_params=pltpu.CompilerParams(
            dimension_semantics=("arbitrary",),
        ),
    )(x2, w_groups, wot, Wgate, bknn, bg)
    return out.reshape(b, s, d)


# speedup vs baseline: 1.2321x; 1.0199x over previous
import jax
import jax.numpy as jnp
import numpy as np
from jax.experimental import pallas as pl
from jax.experimental.pallas import tpu as pltpu

D = 768
H = 12
HD = D // H
GH = 6
NSTEP = H // GH
QBLK = 256


def _dot(a, b, dims):
    return jax.lax.dot_general(a, b, (dims, ((), ())),
                               preferred_element_type=jnp.float32)


def _causal_head_attn(q, k, v):
    n = q.shape[0]
    nblk = n // QBLK
    row = jax.lax.broadcasted_iota(jnp.int32, (QBLK, QBLK), 0)
    col = jax.lax.broadcasted_iota(jnp.int32, (QBLK, QBLK), 1)
    diag_mask = col <= row

    blocks = []
    for i in range(nblk):
        qi = q[i * QBLK:(i + 1) * QBLK, :]
        kd = k[i * QBLK:(i + 1) * QBLK, :]
        vd = v[i * QBLK:(i + 1) * QBLK, :]
        s_diag = _dot(qi, kd, ((1,), (1,)))
        s_diag = jnp.where(diag_mask, s_diag, -jnp.inf)
        if i == 0:
            m = jnp.max(s_diag, axis=1, keepdims=True)
            p_diag = jnp.exp(s_diag - m)
            l = jnp.sum(p_diag, axis=1, keepdims=True)
            acc = _dot(p_diag, vd, ((1,), (0,)))
        else:
            kp = k[:i * QBLK, :]
            vp = v[:i * QBLK, :]
            s_prev = _dot(qi, kp, ((1,), (1,)))
            m = jnp.maximum(jnp.max(s_prev, axis=1, keepdims=True),
                            jnp.max(s_diag, axis=1, keepdims=True))
            p_prev = jnp.exp(s_prev - m)
            p_diag = jnp.exp(s_diag - m)
            l = (jnp.sum(p_prev, axis=1, keepdims=True) +
                 jnp.sum(p_diag, axis=1, keepdims=True))
            acc = (_dot(p_prev, vp, ((1,), (0,))) +
                   _dot(p_diag, vd, ((1,), (0,))))
        blocks.append(acc / l)
    return jnp.concatenate(blocks, axis=0)


def _mha_gate_kernel(x_ref, w_ref, wot_ref, wg_ref, bknn_ref, bg_ref,
                     out_ref):
    t = pl.program_id(0)
    gw = GH * HD

    qkv = _dot(x_ref[...], w_ref[...], ((1,), (1,)))
    scale = 1.0 / np.sqrt(HD)

    heads = []
    for g in range(GH):
        q = qkv[:, g * HD:(g + 1) * HD] * scale
        k = qkv[:, gw + g * HD:gw + (g + 1) * HD]
        v = qkv[:, 2 * gw + g * HD:2 * gw + (g + 1) * HD]
        heads.append(_causal_head_attn(q, k, v))
    attn_grp = jnp.concatenate(heads, axis=1)
    o = _dot(attn_grp, wot_ref[...], ((1,), (0,)))

    @pl.when(t == 0)
    def _():
        out_ref[...] = o

    @pl.when(t > 0)
    def _():
        out_ref[...] += o

    @pl.when(t == NSTEP - 1)
    def _():
        a = out_ref[...]
        wg = wg_ref[...]
        wg1 = wg[:, :D]
        wg2 = wg[:, D:]
        bknn = bknn_ref[...]
        c = jnp.sum(bknn * wg2) + bg_ref[0, 0]
        g_logit = jnp.sum(a * wg1, axis=1, keepdims=True) + c
        g = jax.nn.sigmoid(g_logit)
        out_ref[...] = g * a + (1.0 - g) * bknn


def kernel(x, Wq, Wk, Wv, Wo, Wknn_key, Wknn_proj, b_knn_proj, Wgate, b_gate):
    b, s, d = x.shape
    x2 = x.reshape(s, d)
    w3 = jnp.stack([Wq, Wk, Wv], axis=0)
    w3 = w3.reshape(3, NSTEP, GH * HD, d)
    w_groups = w3.transpose(1, 0, 2, 3).reshape(NSTEP * 3 * GH * HD, d)
    wot = Wo.T
    bknn = b_knn_proj.reshape(1, d)
    bg = b_gate.reshape(1, 1)

    out = pl.pallas_call(
        _mha_gate_kernel,
        grid=(NSTEP,),
        in_specs=[
            pl.BlockSpec((s, d), lambda t: (0, 0)),
            pl.BlockSpec((3 * GH * HD, d), lambda t: (t, 0)),
            pl.BlockSpec((GH * HD, d), lambda t: (t, 0)),
            pl.BlockSpec((1, 2 * d), lambda t: (0, 0)),
            pl.BlockSpec((1, d), lambda t: (0, 0)),
            pl.BlockSpec((1, 1), lambda t: (0, 0)),
        ],
        out_specs=pl.BlockSpec((s, d), lambda t: (0, 0)),
        out_shape=jax.ShapeDtypeStruct((s, d), jnp.float32),
        compiler_params=pltpu.CompilerParams(
            dimension_semantics=("arbitrary",),
            vmem_limit_bytes=100 * 1024 * 1024,
        ),
    )(x2, w_groups, wot, Wgate, bknn, bg)
    return out.reshape(b, s, d)


# speedup vs baseline: 1.2364x; 1.0035x over previous
import jax
import jax.numpy as jnp
import numpy as np
from jax.experimental import pallas as pl
from jax.experimental.pallas import tpu as pltpu

D = 768
H = 12
HD = D // H
GH = 6
NSTEP = H // GH
QBLK = 256


def _dot(a, b, dims):
    return jax.lax.dot_general(a, b, (dims, ((), ())),
                               preferred_element_type=jnp.float32)


def _causal_head_attn(q, k, v):
    n = q.shape[0]
    nblk = n // QBLK
    row = jax.lax.broadcasted_iota(jnp.int32, (QBLK, QBLK), 0)
    col = jax.lax.broadcasted_iota(jnp.int32, (QBLK, QBLK), 1)
    diag_mask = col <= row

    blocks = []
    for i in range(nblk):
        qi = q[i * QBLK:(i + 1) * QBLK, :]
        kd = k[i * QBLK:(i + 1) * QBLK, :]
        vd = v[i * QBLK:(i + 1) * QBLK, :]
        s_diag = _dot(qi, kd, ((1,), (1,)))
        s_diag = jnp.where(diag_mask, s_diag, -jnp.inf)
        if i == 0:
            m = jnp.max(s_diag, axis=1, keepdims=True)
            p_diag = jnp.exp(s_diag - m)
            l = jnp.sum(p_diag, axis=1, keepdims=True)
            acc = _dot(p_diag, vd, ((1,), (0,)))
        else:
            kp = k[:i * QBLK, :]
            vp = v[:i * QBLK, :]
            s_prev = _dot(qi, kp, ((1,), (1,)))
            m = jnp.maximum(jnp.max(s_prev, axis=1, keepdims=True),
                            jnp.max(s_diag, axis=1, keepdims=True))
            p_prev = jnp.exp(s_prev - m)
            p_diag = jnp.exp(s_diag - m)
            l = (jnp.sum(p_prev, axis=1, keepdims=True) +
                 jnp.sum(p_diag, axis=1, keepdims=True))
            acc = (_dot(p_prev, vp, ((1,), (0,))) +
                   _dot(p_diag, vd, ((1,), (0,))))
        blocks.append(acc / l)
    return jnp.concatenate(blocks, axis=0)


def _mha_gate_kernel(x_ref, w_ref, wot_ref, wg_ref, bknn_ref, bg_ref,
                     out_ref):
    t = pl.program_id(0)
    gw = GH * HD

    qkv = _dot(x_ref[...], w_ref[...], ((1,), (1,)))
    scale = 1.0 / np.sqrt(HD)

    heads = []
    for g in range(GH):
        q = qkv[:, g * HD:(g + 1) * HD] * scale
        k = qkv[:, gw + g * HD:gw + (g + 1) * HD]
        v = qkv[:, 2 * gw + g * HD:2 * gw + (g + 1) * HD]
        heads.append(_causal_head_attn(q, k, v))
    attn_grp = jnp.concatenate(heads, axis=1)
    o = _dot(attn_grp, wot_ref[...], ((1,), (0,)))

    @pl.when(t < NSTEP - 1)
    def _():
        @pl.when(t == 0)
        def _():
            out_ref[...] = o

        @pl.when(t > 0)
        def _():
            out_ref[...] += o

    @pl.when(t == NSTEP - 1)
    def _():
        a = out_ref[...] + o
        wg = wg_ref[...]
        wg1 = wg[:, :D]
        wg2 = wg[:, D:]
        bknn = bknn_ref[...]
        c = jnp.sum(bknn * wg2) + bg_ref[0, 0]
        g_logit = jnp.sum(a * wg1, axis=1, keepdims=True) + c
        g = jax.nn.sigmoid(g_logit)
        out_ref[...] = g * a + (1.0 - g) * bknn


def kernel(x, Wq, Wk, Wv, Wo, Wknn_key, Wknn_proj, b_knn_proj, Wgate, b_gate):
    b, s, d = x.shape
    x2 = x.reshape(s, d)
    w3 = jnp.stack([Wq, Wk, Wv], axis=0)
    w3 = w3.reshape(3, NSTEP, GH * HD, d)
    w_groups = w3.transpose(1, 0, 2, 3).reshape(NSTEP * 3 * GH * HD, d)
    wot = Wo.T
    bknn = b_knn_proj.reshape(1, d)
    bg = b_gate.reshape(1, 1)

    out = pl.pallas_call(
        _mha_gate_kernel,
        grid=(NSTEP,),
        in_specs=[
            pl.BlockSpec((s, d), lambda t: (0, 0)),
            pl.BlockSpec((3 * GH * HD, d), lambda t: (t, 0)),
            pl.BlockSpec((GH * HD, d), lambda t: (t, 0)),
            pl.BlockSpec((1, 2 * d), lambda t: (0, 0)),
            pl.BlockSpec((1, d), lambda t: (0, 0)),
            pl.BlockSpec((1, 1), lambda t: (0, 0)),
        ],
        out_specs=pl.BlockSpec((s, d), lambda t: (0, 0)),
        out_shape=jax.ShapeDtypeStruct((s, d), jnp.float32),
        compiler_params=pltpu.CompilerParams(
            dimension_semantics=("arbitrary",),
            vmem_limit_bytes=100 * 1024 * 1024,
        ),
    )(x2, w_groups, wot, Wgate, bknn, bg)
    return out.reshape(b, s, d)
